# baseline (speedup 1.0000x reference)
import functools

import jax
import jax.numpy as jnp
from jax import lax
from jax.experimental import pallas as pl
from jax.experimental.pallas import tpu as pltpu
from jax.experimental.pallas import tpu_sc as plsc

N_TOK = 16384
D = 256
V = 8192
TM = 4096
TW = 2736
V_PAD = 8208
N_WIN = 3
N_TT = N_TOK // TM
N_CH = TW // 8


def _argmin_body(z_ref, z2_ref, e_ref, e2_ref, idx_out, loss_out,
                 accv_s, acci_s, accx_s, lsum_s):
    j = pl.program_id(0)
    i = pl.program_id(1)

    zb = z_ref[...].astype(jnp.bfloat16)
    eb = jnp.bfloat16(-2.0) * e_ref[...].astype(jnp.bfloat16)
    s2 = lax.dot_general(eb, zb, (((1,), (1,)), ((), ())),
                         preferred_element_type=jnp.float32)
    z2 = z2_ref[...]
    m = (z2 + e2_ref[0:8, 0:1]) + s2[0:8, :]
    mi = jnp.zeros((8, TM), jnp.int32)
    for c in range(1, N_CH):
        dch = (z2 + e2_ref[c * 8:(c + 1) * 8, 0:1]) + s2[c * 8:(c + 1) * 8, :]
        lt = dch < m
        m = jnp.where(lt, dch, m)
        mi = jnp.where(lt, jnp.int32(c), mi)

    subl = lax.broadcasted_iota(jnp.int32, (8, TM), 0)
    gidx = mi * 8 + subl
    wmin = jnp.min(m, axis=0, keepdims=True)
    warg = jnp.min(jnp.where(m == wmin, gidx, jnp.int32(2**30)),
                   axis=0, keepdims=True) + j * TW

    row = pl.ds(i, 1)

    @pl.when(j == 0)
    def _first():
        accv_s[row, :] = wmin.astype(jnp.bfloat16).astype(jnp.float32)
        acci_s[row, :] = warg
        accx_s[row, :] = wmin

    @pl.when(j > 0)
    def _fold():
        av = accv_s[row, :]
        ai = acci_s[row, :]
        ax = accx_s[row, :]
        lt = wmin < av
        take = lt | ((wmin == av) & (warg < ai))
        acci_s[row, :] = jnp.where(take, warg, ai)
        accx_s[row, :] = jnp.where(take, wmin, ax)
        accv_s[row, :] = jnp.where(lt, wmin, av).astype(
            jnp.bfloat16).astype(jnp.float32)

    @pl.when(j == N_WIN - 1)
    def _finish():
        idx_out[...] = jnp.reshape(acci_s[row, :], (TM,))

        @pl.when(i == 0)
        def _z():
            lsum_s[0] = 0.0
        lsum_s[0] += jnp.sum(accx_s[row, :])

        @pl.when(i == N_TT - 1)
        def _w():
            loss_out[0, 0] = 2.0 * lsum_s[0] / jnp.float32(N_TOK * D)


def _argmin_call(z_bf, z2r, e_bf, e2c):
    return pl.pallas_call(
        _argmin_body,
        grid=(N_WIN, N_TT),
        in_specs=[
            pl.BlockSpec((TM, D), lambda j, i: (i, 0)),
            pl.BlockSpec((1, TM), lambda j, i: (0, i)),
            pl.BlockSpec((TW, D), lambda j, i: (j, 0)),
            pl.BlockSpec((TW, 1), lambda j, i: (j, 0)),
        ],
        out_specs=[
            pl.BlockSpec((TM,), lambda j, i: (i,)),
            pl.BlockSpec(memory_space=pltpu.SMEM),
        ],
        out_shape=[
            jax.ShapeDtypeStruct((N_TOK,), jnp.int32),
            jax.ShapeDtypeStruct((1, 1), jnp.float32),
        ],
        scratch_shapes=[
            pltpu.VMEM((N_TT, TM), jnp.float32),
            pltpu.VMEM((N_TT, TM), jnp.int32),
            pltpu.VMEM((N_TT, TM), jnp.float32),
            pltpu.SMEM((1,), jnp.float32),
        ],
    )(z_bf, z2r, e_bf, e2c)


_NW = 32
_BPW = N_TOK // _NW
_CHUNK = 128
_NCHUNK = _BPW // _CHUNK


def _gather_body(idx_hbm, table_hbm, out_hbm, idx_v, buf0, buf1, sem0, sem1):
    wid = lax.axis_index("s") * 2 + lax.axis_index("c")
    base = wid * _BPW
    pltpu.sync_copy(idx_hbm.at[pl.ds(base, _BPW)], idx_v)
    bufs = (buf0, buf1)
    sems = (sem0, sem1)

    def start(c):
        return pltpu.async_copy(
            table_hbm.at[idx_v.at[pl.ds(c * _CHUNK, _CHUNK)]],
            bufs[c % 2], sems[c % 2])

    cur = start(0)
    for c in range(_NCHUNK):
        nxt = start(c + 1) if c + 1 < _NCHUNK else None
        cur.wait()
        pltpu.sync_copy(bufs[c % 2],
                        out_hbm.at[pl.ds(base + c * _CHUNK, _CHUNK)])
        cur = nxt


def _gather_call(idx, emb):
    mesh = plsc.VectorSubcoreMesh(core_axis_name="c", subcore_axis_name="s")
    k = functools.partial(
        pl.kernel, mesh=mesh,
        out_type=jax.ShapeDtypeStruct((N_TOK, D), jnp.float32),
        scratch_types=[
            pltpu.VMEM((_BPW,), jnp.int32),
            pltpu.VMEM((_CHUNK, D), jnp.float32),
            pltpu.VMEM((_CHUNK, D), jnp.float32),
            pltpu.SemaphoreType.DMA,
            pltpu.SemaphoreType.DMA,
        ],
    )(_gather_body)
    return k(idx, emb)


def kernel(z_e, embedding_weight):
    z2r = jnp.sum(z_e ** 2, axis=-1).reshape(1, N_TOK)
    e2 = jnp.sum(embedding_weight ** 2, axis=1)

    z_flat = z_e.reshape(N_TOK, D)
    e_pad = jnp.pad(embedding_weight, ((0, V_PAD - V), (0, 0)))
    e2c = jnp.pad(e2, (0, V_PAD - V),
                  constant_values=jnp.inf).reshape(V_PAD, 1)

    idx, loss = _argmin_call(z_flat, z2r, e_pad, e2c)
    z_q = _gather_call(idx, embedding_weight)
    return (z_e, jnp.reshape(loss, ()), z_q.reshape(z_e.shape))


# speedup vs baseline: 1.0740x; 1.0740x over previous
import functools

import jax
import jax.numpy as jnp
from jax import lax
from jax.experimental import pallas as pl
from jax.experimental.pallas import tpu as pltpu
from jax.experimental.pallas import tpu_sc as plsc

N_TOK = 16384
D = 256
V = 8192
TM = 4096
TW = 2736
V_PAD = 8208
N_WIN = 3
N_TT = N_TOK // TM
N_CH = TW // 8


def _argmin_body(z_ref, z2_ref, e_ref, e2_ref, idx_out, loss_out,
                 accv_s, acci_s, accx_s, lsum_s):
    j = pl.program_id(0)
    i = pl.program_id(1)

    zb = z_ref[...]
    eb = e_ref[...]
    s2 = lax.dot_general(eb, zb, (((1,), (1,)), ((), ())),
                         preferred_element_type=jnp.float32)
    z2 = z2_ref[...]
    m = (z2 + e2_ref[0:8, 0:1]) + s2[0:8, :]
    mi = jnp.zeros((8, TM), jnp.int32)
    for c in range(1, N_CH):
        dch = (z2 + e2_ref[c * 8:(c + 1) * 8, 0:1]) + s2[c * 8:(c + 1) * 8, :]
        lt = dch < m
        m = jnp.where(lt, dch, m)
        mi = jnp.where(lt, jnp.int32(c), mi)

    subl = lax.broadcasted_iota(jnp.int32, (8, TM), 0)
    gidx = mi * 8 + subl
    wmin = jnp.min(m, axis=0, keepdims=True)
    warg = jnp.min(jnp.where(m == wmin, gidx, jnp.int32(2**30)),
                   axis=0, keepdims=True) + j * TW

    row = pl.ds(i, 1)

    @pl.when(j == 0)
    def _first():
        accv_s[row, :] = wmin.astype(jnp.bfloat16).astype(jnp.float32)
        acci_s[row, :] = warg
        accx_s[row, :] = wmin

    @pl.when(j > 0)
    def _fold():
        av = accv_s[row, :]
        ai = acci_s[row, :]
        ax = accx_s[row, :]
        lt = wmin < av
        take = lt | ((wmin == av) & (warg < ai))
        acci_s[row, :] = jnp.where(take, warg, ai)
        accx_s[row, :] = jnp.where(take, wmin, ax)
        accv_s[row, :] = jnp.where(lt, wmin, av).astype(
            jnp.bfloat16).astype(jnp.float32)

    @pl.when(j == N_WIN - 1)
    def _finish():
        idx_out[...] = jnp.reshape(acci_s[row, :], (TM,))

        @pl.when(i == 0)
        def _z():
            lsum_s[0] = 0.0
        lsum_s[0] += jnp.sum(accx_s[row, :])

        @pl.when(i == N_TT - 1)
        def _w():
            loss_out[0, 0] = 2.0 * lsum_s[0] / jnp.float32(N_TOK * D)


def _argmin_call(z_bf, z2r, e_bf, e2c):
    return pl.pallas_call(
        _argmin_body,
        grid=(N_WIN, N_TT),
        in_specs=[
            pl.BlockSpec((TM, D), lambda j, i: (i, 0)),
            pl.BlockSpec((1, TM), lambda j, i: (0, i)),
            pl.BlockSpec((TW, D), lambda j, i: (j, 0)),
            pl.BlockSpec((TW, 1), lambda j, i: (j, 0)),
        ],
        out_specs=[
            pl.BlockSpec((TM,), lambda j, i: (i,)),
            pl.BlockSpec(memory_space=pltpu.SMEM),
        ],
        out_shape=[
            jax.ShapeDtypeStruct((N_TOK,), jnp.int32),
            jax.ShapeDtypeStruct((1, 1), jnp.float32),
        ],
        scratch_shapes=[
            pltpu.VMEM((N_TT, TM), jnp.float32),
            pltpu.VMEM((N_TT, TM), jnp.int32),
            pltpu.VMEM((N_TT, TM), jnp.float32),
            pltpu.SMEM((1,), jnp.float32),
        ],
    )(z_bf, z2r, e_bf, e2c)


_NW = 32
_BPW = N_TOK // _NW
_CHUNK = 128
_NCHUNK = _BPW // _CHUNK


def _gather_body(idx_hbm, table_hbm, out_hbm, idx_v, buf0, buf1, sem0, sem1):
    wid = lax.axis_index("s") * 2 + lax.axis_index("c")
    base = wid * _BPW
    pltpu.sync_copy(idx_hbm.at[pl.ds(base, _BPW)], idx_v)
    bufs = (buf0, buf1)
    sems = (sem0, sem1)

    def start(c):
        return pltpu.async_copy(
            table_hbm.at[idx_v.at[pl.ds(c * _CHUNK, _CHUNK)]],
            bufs[c % 2], sems[c % 2])

    cur = start(0)
    for c in range(_NCHUNK):
        nxt = start(c + 1) if c + 1 < _NCHUNK else None
        cur.wait()
        pltpu.sync_copy(bufs[c % 2],
                        out_hbm.at[pl.ds(base + c * _CHUNK, _CHUNK)])
        cur = nxt


def _gather_call(idx, emb):
    mesh = plsc.VectorSubcoreMesh(core_axis_name="c", subcore_axis_name="s")
    k = functools.partial(
        pl.kernel, mesh=mesh,
        out_type=jax.ShapeDtypeStruct((N_TOK, D), jnp.float32),
        scratch_types=[
            pltpu.VMEM((_BPW,), jnp.int32),
            pltpu.VMEM((_CHUNK, D), jnp.float32),
            pltpu.VMEM((_CHUNK, D), jnp.float32),
            pltpu.SemaphoreType.DMA,
            pltpu.SemaphoreType.DMA,
        ],
    )(_gather_body)
    return k(idx, emb)


def kernel(z_e, embedding_weight):
    z2r = jnp.sum(z_e ** 2, axis=-1).reshape(1, N_TOK)
    e2 = jnp.sum(embedding_weight ** 2, axis=1)

    z_bf = z_e.reshape(N_TOK, D).astype(jnp.bfloat16)
    e_bf = jnp.pad((-2.0 * embedding_weight).astype(jnp.bfloat16),
                   ((0, V_PAD - V), (0, 0)))
    e2c = jnp.pad(e2, (0, V_PAD - V),
                  constant_values=jnp.inf).reshape(V_PAD, 1)

    idx, loss = _argmin_call(z_bf, z2r, e_bf, e2c)
    z_q = _gather_call(idx, embedding_weight)
    return (z_e, jnp.reshape(loss, ()), z_q.reshape(z_e.shape))
